# baseline (speedup 1.0000x reference)
import functools

import jax
import jax.numpy as jnp
from jax import lax
from jax.experimental import pallas as pl
from jax.experimental.pallas import tpu as pltpu
from jax.experimental.pallas import tpu_sc as plsc

N = 100000
D = 92
TW = 96
NROWS = 100
EAN_LEN = 89
EAN_PAD = 96
CHUNK = 160
NCHUNKS = N // CHUNK
GROUPS = CHUNK // 16

_info = plsc.get_sparse_core_info()
_NC, _NS, _L = _info.num_cores, _info.num_subcores, _info.num_lanes
NW = _NC * _NS
NBUF = 2
ITERS = -(-NCHUNKS // NW)
ITERS += ITERS % NBUF

_mesh = plsc.VectorSubcoreMesh(core_axis_name="c", subcore_axis_name="s")


@functools.partial(
    pl.kernel,
    mesh=_mesh,
    out_type=jax.ShapeDtypeStruct((N, D), jnp.float32),
    compiler_params=pltpu.CompilerParams(
        needs_layout_passes=False, use_tc_tiling_on_sc=True),
    scratch_types=[
        pltpu.VMEM((EAN_PAD,), jnp.int32),
        pltpu.VMEM((CHUNK,), jnp.int32),
        pltpu.VMEM((NROWS * TW,), jnp.float32),
        pltpu.VMEM((NBUF * CHUNK, D), jnp.float32),
        pltpu.SemaphoreType.DMA,
        pltpu.SemaphoreType.DMA,
    ],
)
def _embed_kernel(idx_hbm, ean_hbm, tab_hbm, out_hbm,
                  ean_v, idx_v, tab_v, rows_v, ssem0, ssem1):
    wid = lax.axis_index("s") * _NC + lax.axis_index("c")
    pltpu.sync_copy(ean_hbm, ean_v)
    pltpu.sync_copy(tab_hbm, tab_v)
    ssems = [ssem0, ssem1]
    lanes = lax.iota(jnp.int32, _L)

    @pl.loop(0, ITERS, step=NBUF)
    def _iter(it):
        for b in range(NBUF):
            g = (it + b) * NW + wid
            gc = jnp.minimum(g, NCHUNKS - 1)

            @pl.when(it + b >= NBUF)
            def _drain():
                pltpu.make_async_copy(
                    rows_v.at[pl.ds(b * CHUNK, CHUNK)],
                    out_hbm.at[pl.ds(gc * CHUNK, CHUNK)],
                    ssems[b]).wait()

            @pl.when(g < NCHUNKS)
            def _chunk():
                pltpu.sync_copy(idx_hbm.at[pl.ds(g * CHUNK, CHUNK)], idx_v)
                for j in range(GROUPS):
                    a = idx_v[pl.ds(j * 16, 16)]
                    t = plsc.load_gather(ean_v, [a])
                    t96 = t * TW
                    r16 = b * CHUNK + j * 16 + lanes
                    for c in range(D):
                        v = plsc.load_gather(tab_v, [t96 + c])
                        plsc.store_scatter(
                            rows_v, [r16, jnp.full((16,), c, jnp.int32)], v)
                pltpu.make_async_copy(
                    rows_v.at[pl.ds(b * CHUNK, CHUNK)],
                    out_hbm.at[pl.ds(g * CHUNK, CHUNK)],
                    ssems[b]).start()

    for b in range(NBUF):
        last_g = (ITERS - NBUF + b) * NW + wid

        @pl.when(last_g < NCHUNKS)
        def _final(b=b, last_g=last_g):
            gc = jnp.minimum(last_g, NCHUNKS - 1)
            pltpu.make_async_copy(
                rows_v.at[pl.ds(b * CHUNK, CHUNK)],
                out_hbm.at[pl.ds(gc * CHUNK, CHUNK)],
                ssems[b]).wait()


def kernel(indices, element_atomic_numbers, cgcnn_table):
    idx = jnp.asarray(indices, jnp.int32)
    ean = jnp.concatenate([
        jnp.asarray(element_atomic_numbers, jnp.int32),
        jnp.zeros((EAN_PAD - EAN_LEN,), jnp.int32),
    ])
    table = jnp.asarray(cgcnn_table, jnp.float32)
    tab96 = jnp.pad(table, ((0, 0), (0, TW - D))).reshape(NROWS * TW)
    return _embed_kernel(idx, ean, tab96)


# speedup vs baseline: 2.0675x; 2.0675x over previous
import functools

import jax
import jax.numpy as jnp
from jax import lax
from jax.experimental import pallas as pl
from jax.experimental.pallas import tpu as pltpu
from jax.experimental.pallas import tpu_sc as plsc

N = 100000
D = 92
TW = 97
NROWS = 100
EAN_LEN = 89
EAN_PAD = 96
CHUNK = 160
NCHUNKS = N // CHUNK
GROUPS = CHUNK // 16

_info = plsc.get_sparse_core_info()
_NC, _NS, _L = _info.num_cores, _info.num_subcores, _info.num_lanes
NW = _NC * _NS
NBUF = 2
ITERS = -(-NCHUNKS // NW)
ITERS += ITERS % NBUF

_mesh = plsc.VectorSubcoreMesh(core_axis_name="c", subcore_axis_name="s")

_BCAST_DNUMS = lax.GatherDimensionNumbers(
    offset_dims=(), collapsed_slice_dims=(0,), start_index_map=(0,))


def _lane_broadcast(vec, rr):
    idx = jnp.full((16, 1), rr, jnp.int32)
    return lax.gather(vec, idx, _BCAST_DNUMS, (1,),
                      mode=lax.GatherScatterMode.PROMISE_IN_BOUNDS)


@functools.partial(
    pl.kernel,
    mesh=_mesh,
    out_type=jax.ShapeDtypeStruct((N, D), jnp.float32),
    compiler_params=pltpu.CompilerParams(
        needs_layout_passes=False, use_tc_tiling_on_sc=True),
    scratch_types=[
        pltpu.VMEM((EAN_PAD,), jnp.int32),
        pltpu.VMEM((CHUNK,), jnp.int32),
        pltpu.VMEM((NROWS * TW,), jnp.float32),
        pltpu.VMEM((NBUF * CHUNK, D), jnp.float32),
        pltpu.SemaphoreType.DMA,
        pltpu.SemaphoreType.DMA,
    ],
)
def _embed_kernel(idx_hbm, ean_hbm, tab_hbm, out_hbm,
                  ean_v, idx_v, tab_v, rows_v, ssem0, ssem1):
    wid = lax.axis_index("s") * _NC + lax.axis_index("c")
    pltpu.sync_copy(ean_hbm, ean_v)
    pltpu.sync_copy(tab_hbm, tab_v)
    ssems = [ssem0, ssem1]
    lanes = lax.iota(jnp.int32, _L)

    @pl.loop(0, ITERS, step=NBUF)
    def _iter(it):
        for b in range(NBUF):
            g = (it + b) * NW + wid
            gc = jnp.minimum(g, NCHUNKS - 1)

            @pl.when(it + b >= NBUF)
            def _drain():
                pltpu.make_async_copy(
                    rows_v.at[pl.ds(b * CHUNK, CHUNK)],
                    out_hbm.at[pl.ds(gc * CHUNK, CHUNK)],
                    ssems[b]).wait()

            @pl.when(g < NCHUNKS)
            def _chunk():
                pltpu.sync_copy(idx_hbm.at[pl.ds(g * CHUNK, CHUNK)], idx_v)
                for j in range(GROUPS):
                    a = idx_v[pl.ds(j * 16, 16)]
                    t = plsc.load_gather(ean_v, [a])
                    ob = t * TW
                    for rr in range(16):
                        row = b * CHUNK + j * 16 + rr
                        obr = _lane_broadcast(ob, rr) + lanes
                        for off in (0, 16, 32, 48, 64, 76):
                            v = plsc.load_gather(tab_v, [obr + off])
                            rows_v[row, pl.ds(off, 16)] = v
                pltpu.make_async_copy(
                    rows_v.at[pl.ds(b * CHUNK, CHUNK)],
                    out_hbm.at[pl.ds(g * CHUNK, CHUNK)],
                    ssems[b]).start()

    for b in range(NBUF):
        last_g = (ITERS - NBUF + b) * NW + wid

        @pl.when(last_g < NCHUNKS)
        def _final(b=b, last_g=last_g):
            gc = jnp.minimum(last_g, NCHUNKS - 1)
            pltpu.make_async_copy(
                rows_v.at[pl.ds(b * CHUNK, CHUNK)],
                out_hbm.at[pl.ds(gc * CHUNK, CHUNK)],
                ssems[b]).wait()


def kernel(indices, element_atomic_numbers, cgcnn_table):
    idx = jnp.asarray(indices, jnp.int32)
    ean = jnp.concatenate([
        jnp.asarray(element_atomic_numbers, jnp.int32),
        jnp.zeros((EAN_PAD - EAN_LEN,), jnp.int32),
    ])
    table = jnp.asarray(cgcnn_table, jnp.float32)
    tab96 = jnp.pad(table, ((0, 0), (0, TW - D))).reshape(NROWS * TW)
    return _embed_kernel(idx, ean, tab96)


# speedup vs baseline: 2.3082x; 1.1164x over previous
import functools

import jax
import jax.numpy as jnp
from jax import lax
from jax.experimental import pallas as pl
from jax.experimental.pallas import tpu as pltpu
from jax.experimental.pallas import tpu_sc as plsc

N = 100000
D = 92
TW = 97
NROWS = 100
EAN_LEN = 89
EAN_PAD = 96
CHUNK = 160
NCHUNKS = N // CHUNK
GROUPS = CHUNK // 16

_info = plsc.get_sparse_core_info()
_NC, _NS, _L = _info.num_cores, _info.num_subcores, _info.num_lanes
NW = _NC * _NS
NBUF = 2
ITERS = -(-NCHUNKS // NW)
ITERS += ITERS % NBUF

_mesh = plsc.VectorSubcoreMesh(core_axis_name="c", subcore_axis_name="s")

_BCAST_DNUMS = lax.GatherDimensionNumbers(
    offset_dims=(), collapsed_slice_dims=(0,), start_index_map=(0,))


def _lane_broadcast(vec, rr):
    idx = jnp.full((16, 1), rr, jnp.int32)
    return lax.gather(vec, idx, _BCAST_DNUMS, (1,),
                      mode=lax.GatherScatterMode.PROMISE_IN_BOUNDS)


@functools.partial(
    pl.kernel,
    mesh=_mesh,
    out_type=jax.ShapeDtypeStruct((N, D), jnp.float32),
    compiler_params=pltpu.CompilerParams(
        needs_layout_passes=False, use_tc_tiling_on_sc=True),
    scratch_types=[
        pltpu.VMEM((EAN_PAD,), jnp.int32),
        pltpu.VMEM((CHUNK,), jnp.int32),
        pltpu.VMEM((NROWS * TW,), jnp.float32),
        pltpu.VMEM((NBUF * CHUNK, D), jnp.float32),
        pltpu.SemaphoreType.DMA,
        pltpu.SemaphoreType.DMA,
    ],
)
def _embed_kernel(idx_hbm, ean_hbm, tab_hbm, out_hbm,
                  ean_v, idx_v, tab_v, rows_v, ssem0, ssem1):
    wid = lax.axis_index("s") * _NC + lax.axis_index("c")
    pltpu.sync_copy(ean_hbm, ean_v)
    pltpu.sync_copy(tab_hbm, tab_v)
    ssems = [ssem0, ssem1]
    lanes = lax.iota(jnp.int32, _L)

    @pl.loop(0, ITERS, step=NBUF)
    def _iter(it):
        for b in range(NBUF):
            g = (it + b) * NW + wid
            gc = jnp.minimum(g, NCHUNKS - 1)

            @pl.when(it + b >= NBUF)
            def _drain():
                pltpu.make_async_copy(
                    rows_v.at[pl.ds(b * CHUNK, CHUNK)],
                    out_hbm.at[pl.ds(gc * CHUNK, CHUNK)],
                    ssems[b]).wait()

            @pl.when(g < NCHUNKS)
            def _chunk():
                pltpu.sync_copy(idx_hbm.at[pl.ds(g * CHUNK, CHUNK)], idx_v)
                for j in range(GROUPS):
                    a = idx_v[pl.ds(j * 16, 16)]
                    t = plsc.load_gather(ean_v, [a])
                    ob = t * TW
                    for rr in range(16):
                        row = b * CHUNK + j * 16 + rr
                        obr = _lane_broadcast(ob, rr) + lanes
                        for off in (0, 16, 32, 48, 64):
                            v = plsc.load_gather(
                                tab_v.at[pl.ds(off, NROWS * TW - off)],
                                [obr])
                            rows_v[row, pl.ds(off, 16)] = v
                        v = plsc.load_gather(tab_v, [obr + 76])
                        rows_v[row, pl.ds(76, 16)] = v
                pltpu.make_async_copy(
                    rows_v.at[pl.ds(b * CHUNK, CHUNK)],
                    out_hbm.at[pl.ds(g * CHUNK, CHUNK)],
                    ssems[b]).start()

    for b in range(NBUF):
        last_g = (ITERS - NBUF + b) * NW + wid

        @pl.when(last_g < NCHUNKS)
        def _final(b=b, last_g=last_g):
            gc = jnp.minimum(last_g, NCHUNKS - 1)
            pltpu.make_async_copy(
                rows_v.at[pl.ds(b * CHUNK, CHUNK)],
                out_hbm.at[pl.ds(gc * CHUNK, CHUNK)],
                ssems[b]).wait()


def kernel(indices, element_atomic_numbers, cgcnn_table):
    idx = jnp.asarray(indices, jnp.int32)
    ean = jnp.concatenate([
        jnp.asarray(element_atomic_numbers, jnp.int32),
        jnp.zeros((EAN_PAD - EAN_LEN,), jnp.int32),
    ])
    table = jnp.asarray(cgcnn_table, jnp.float32)
    tab96 = jnp.pad(table, ((0, 0), (0, TW - D))).reshape(NROWS * TW)
    return _embed_kernel(idx, ean, tab96)


# speedup vs baseline: 2.5165x; 1.0902x over previous
import functools

import jax
import jax.numpy as jnp
from jax import lax
from jax.experimental import pallas as pl
from jax.experimental.pallas import tpu as pltpu
from jax.experimental.pallas import tpu_sc as plsc

N = 100000
D = 92
TW = 97
NROWS = 100
EAN_LEN = 89
EAN_PAD = 96
CHUNK = 160
NCHUNKS = N // CHUNK
GROUPS = CHUNK // 16

_info = plsc.get_sparse_core_info()
_NC, _NS, _L = _info.num_cores, _info.num_subcores, _info.num_lanes
NW = _NC * _NS
NBUF = 2
ITERS = -(-NCHUNKS // NW)
ITERS += ITERS % NBUF

_mesh = plsc.VectorSubcoreMesh(core_axis_name="c", subcore_axis_name="s")

_BCAST_DNUMS = lax.GatherDimensionNumbers(
    offset_dims=(), collapsed_slice_dims=(0,), start_index_map=(0,))


def _lane_broadcast(vec, rr):
    idx = jnp.full((16, 1), rr, jnp.int32)
    return lax.gather(vec, idx, _BCAST_DNUMS, (1,),
                      mode=lax.GatherScatterMode.PROMISE_IN_BOUNDS)


@functools.partial(
    pl.kernel,
    mesh=_mesh,
    out_type=jax.ShapeDtypeStruct((N, D), jnp.float32),
    compiler_params=pltpu.CompilerParams(
        needs_layout_passes=False, use_tc_tiling_on_sc=True),
    scratch_types=[
        pltpu.VMEM((EAN_PAD,), jnp.int32),
        pltpu.VMEM((NBUF * CHUNK,), jnp.int32),
        pltpu.VMEM((NROWS * TW,), jnp.float32),
        pltpu.VMEM((NBUF * CHUNK, D), jnp.float32),
        pltpu.SemaphoreType.DMA,
        pltpu.SemaphoreType.DMA,
        pltpu.SemaphoreType.DMA,
        pltpu.SemaphoreType.DMA,
    ],
)
def _embed_kernel(idx_hbm, ean_hbm, tab_hbm, out_hbm,
                  ean_v, idx_v, tab_v, rows_v, ssem0, ssem1, isem0, isem1):
    wid = lax.axis_index("s") * _NC + lax.axis_index("c")
    pltpu.sync_copy(ean_hbm, ean_v)
    pltpu.sync_copy(tab_hbm, tab_v)
    ssems = [ssem0, ssem1]
    isems = [isem0, isem1]
    lanes = lax.iota(jnp.int32, _L)

    for b in range(NBUF):
        g0 = b * NW + wid

        @pl.when(g0 < NCHUNKS)
        def _prime(b=b, g0=g0):
            pltpu.make_async_copy(
                idx_hbm.at[pl.ds(g0 * CHUNK, CHUNK)],
                idx_v.at[pl.ds(b * CHUNK, CHUNK)], isems[b]).start()

    @pl.loop(0, ITERS, step=NBUF)
    def _iter(it):
        for b in range(NBUF):
            g = (it + b) * NW + wid
            gc = jnp.minimum(g, NCHUNKS - 1)

            @pl.when(it + b >= NBUF)
            def _drain():
                pltpu.make_async_copy(
                    rows_v.at[pl.ds(b * CHUNK, CHUNK)],
                    out_hbm.at[pl.ds(gc * CHUNK, CHUNK)],
                    ssems[b]).wait()

            @pl.when(g < NCHUNKS)
            def _chunk():
                pltpu.make_async_copy(
                    idx_hbm.at[pl.ds(gc * CHUNK, CHUNK)],
                    idx_v.at[pl.ds(b * CHUNK, CHUNK)], isems[b]).wait()
                for j in range(GROUPS):
                    a = idx_v[pl.ds(b * CHUNK + j * 16, 16)]
                    t = plsc.load_gather(ean_v, [a])
                    ob = t * TW
                    for rr in range(16):
                        row = b * CHUNK + j * 16 + rr
                        obr = _lane_broadcast(ob, rr) + lanes
                        for off in (0, 16, 32, 48, 64):
                            v = plsc.load_gather(
                                tab_v.at[pl.ds(off, NROWS * TW - off)],
                                [obr])
                            rows_v[row, pl.ds(off, 16)] = v
                        v = plsc.load_gather(tab_v, [obr + 76])
                        rows_v[row, pl.ds(76, 16)] = v
                gn = g + NBUF * NW

                @pl.when(gn < NCHUNKS)
                def _prefetch():
                    pltpu.make_async_copy(
                        idx_hbm.at[pl.ds(gn * CHUNK, CHUNK)],
                        idx_v.at[pl.ds(b * CHUNK, CHUNK)], isems[b]).start()

                pltpu.make_async_copy(
                    rows_v.at[pl.ds(b * CHUNK, CHUNK)],
                    out_hbm.at[pl.ds(g * CHUNK, CHUNK)],
                    ssems[b]).start()

    for b in range(NBUF):
        last_g = (ITERS - NBUF + b) * NW + wid

        @pl.when(last_g < NCHUNKS)
        def _final(b=b, last_g=last_g):
            gc = jnp.minimum(last_g, NCHUNKS - 1)
            pltpu.make_async_copy(
                rows_v.at[pl.ds(b * CHUNK, CHUNK)],
                out_hbm.at[pl.ds(gc * CHUNK, CHUNK)],
                ssems[b]).wait()


def kernel(indices, element_atomic_numbers, cgcnn_table):
    idx = jnp.asarray(indices, jnp.int32)
    ean = jnp.concatenate([
        jnp.asarray(element_atomic_numbers, jnp.int32),
        jnp.zeros((EAN_PAD - EAN_LEN,), jnp.int32),
    ])
    table = jnp.asarray(cgcnn_table, jnp.float32)
    tab96 = jnp.pad(table, ((0, 0), (0, TW - D))).reshape(NROWS * TW)
    return _embed_kernel(idx, ean, tab96)
